# baseline (speedup 1.0000x reference)
import jax
import jax.numpy as jnp
from jax import lax
from jax.experimental import pallas as pl
from jax.experimental.pallas import tpu as pltpu, tpu_sc as plsc

_CLASSNUM = 70722
_BATCH = 4096
_EPS = 0.001

_NS = 16
_L = 16
_C_PAD = 70728
_CHUNK = _BATCH // _NS
_HALF = _CHUNK // 2

_FN = 0
_SQ2 = 2
_ONES, _ZEROS = 4, 5
_CNT, _SUM, _SQ, _RES = 6, 8, 10, 12


def _newton_rsqrt(v):
    i = lax.bitcast_convert_type(v, jnp.int32)
    i = jnp.int32(0x5F3759DF) - lax.shift_right_logical(i, 1)
    y = lax.bitcast_convert_type(i, jnp.float32)
    for _ in range(3):
        y = y * (1.5 - 0.5 * v * y * y)
    return y


def _sc_body(label_hbm, fn_hbm, out_hbm,
             lab2, buf, sem_a, sem_b, sem_c,
             counts_sh, sums_sh, sqs_sh):
    sid = lax.axis_index("s")

    lab_d = []
    fn_d = []
    for j in range(2):
        base = sid * _CHUNK + j * _HALF
        lab_d.append(pltpu.async_copy(
            label_hbm.at[pl.ds(base, _HALF)], lab2.at[j], sem_a))
        fn_d.append(pltpu.async_copy(
            fn_hbm.at[pl.ds(base, _HALF)], buf.at[_FN + j], sem_b))

    def fill(k, _):
        sl = pl.ds(k * _L, _L)
        buf[_ONES, sl] = jnp.full((_L,), 1.0, jnp.float32)
        buf[_ZEROS, sl] = jnp.full((_L,), 0.0, jnp.float32)
        return 0
    lax.fori_loop(0, _HALF // _L, fill, 0)

    for d in lab_d:
        d.wait()

    zero_d = []
    for j in range(2):
        idx = lab2.at[j]
        zero_d.append(pltpu.async_copy(buf.at[_ZEROS], counts_sh.at[idx], sem_c))
        zero_d.append(pltpu.async_copy(buf.at[_ZEROS], sums_sh.at[idx], sem_c))
        zero_d.append(pltpu.async_copy(buf.at[_ZEROS], sqs_sh.at[idx], sem_c))

    for d in fn_d:
        d.wait()

    def square(k, _):
        sl = pl.ds(k * _L, _L)
        for j in range(2):
            f = buf[_FN + j, sl]
            buf[_SQ2 + j, sl] = f * f
        return 0
    lax.fori_loop(0, _HALF // _L, square, 0)

    for d in zero_d:
        d.wait()
    plsc.subcore_barrier()

    add_d = []
    for j in range(2):
        idx = lab2.at[j]
        add_d.append(pltpu.async_copy(
            buf.at[_ONES], counts_sh.at[idx], sem_a, add=True))
        add_d.append(pltpu.async_copy(
            buf.at[_FN + j], sums_sh.at[idx], sem_a, add=True))
        add_d.append(pltpu.async_copy(
            buf.at[_SQ2 + j], sqs_sh.at[idx], sem_a, add=True))
    for d in add_d:
        d.wait()
    plsc.subcore_barrier()

    gat_d = []
    for j in range(2):
        idx = lab2.at[j]
        gat_d.append(pltpu.async_copy(counts_sh.at[idx], buf.at[_CNT + j], sem_b))
        gat_d.append(pltpu.async_copy(sums_sh.at[idx], buf.at[_SUM + j], sem_b))
        gat_d.append(pltpu.async_copy(sqs_sh.at[idx], buf.at[_SQ + j], sem_b))
    for d in gat_d:
        d.wait()

    def norm(k, _):
        sl = pl.ds(k * _L, _L)
        for j in range(2):
            cnt = buf[_CNT + j, sl]
            s = buf[_SUM + j, sl]
            q = buf[_SQ + j, sl]
            f = buf[_FN + j, sl]
            mean = s / jnp.maximum(cnt, 1.0)
            var = (q - cnt * mean * mean) / jnp.maximum(cnt - 1.0, 1.0)
            var = jnp.maximum(var, 0.0)
            y = _newton_rsqrt(jnp.maximum(var, 1e-30))
            std = var * y
            d = f - mean
            buf[_RES + j, sl] = jnp.where(cnt > 2.0, d / (std + _EPS), d / 20.0)
        return 0
    lax.fori_loop(0, _HALF // _L, norm, 0)

    for j in range(2):
        pltpu.sync_copy(buf.at[_RES + j],
                        out_hbm.at[pl.ds(sid * _CHUNK + j * _HALF, _HALF)])


@jax.jit
def _lamaface_sc(label, fn):
    mesh = plsc.VectorSubcoreMesh(core_axis_name="c", subcore_axis_name="s", num_cores=1)
    run = pl.kernel(
        _sc_body,
        out_type=jax.ShapeDtypeStruct((_BATCH,), jnp.float32),
        mesh=mesh,
        scratch_types=[
            pltpu.VMEM((2, _HALF), jnp.int32),
            pltpu.VMEM((14, _HALF), jnp.float32),
            pltpu.SemaphoreType.DMA,
            pltpu.SemaphoreType.DMA,
            pltpu.SemaphoreType.DMA,
            pltpu.VMEM_SHARED((_C_PAD,), jnp.float32),
            pltpu.VMEM_SHARED((_C_PAD,), jnp.float32),
            pltpu.VMEM_SHARED((_C_PAD,), jnp.float32),
        ],
    )
    return run(label, fn)


def kernel(feature_norm, label, kernel):
    del kernel
    res = _lamaface_sc(label, feature_norm[:, 0])
    return res[:, None]


# speedup vs baseline: 1.0048x; 1.0048x over previous
import jax
import jax.numpy as jnp
from jax import lax
from jax.experimental import pallas as pl
from jax.experimental.pallas import tpu as pltpu, tpu_sc as plsc

_CLASSNUM = 70722
_BATCH = 4096
_EPS = 0.001

_NS = 16
_L = 16
_C_PAD = 70728
_CHUNK = _BATCH // _NS
_HALF = _CHUNK // 2

_FN = 0
_SQ2 = 2
_ONES, _ZEROS = 4, 5
_CNT, _SUM, _SQ, _RES = 6, 8, 10, 12


def _newton_rsqrt(v):
    i = lax.bitcast_convert_type(v, jnp.int32)
    i = jnp.int32(0x5F3759DF) - lax.shift_right_logical(i, 1)
    y = lax.bitcast_convert_type(i, jnp.float32)
    for _ in range(3):
        y = y * (1.5 - 0.5 * v * y * y)
    return y


def _sc_body(label_hbm, fn_hbm, out_hbm,
             lab2, buf, sem_a, sem_b, sem_c,
             counts_sh, sums_sh, sqs_sh):
    sid = lax.axis_index("s")

    lab_d = []
    fn_d = []
    for j in range(2):
        base = sid * _CHUNK + j * _HALF
        lab_d.append(pltpu.async_copy(
            label_hbm.at[pl.ds(base, _HALF)], lab2.at[j], sem_a))
        fn_d.append(pltpu.async_copy(
            fn_hbm.at[pl.ds(base, _HALF)], buf.at[_FN + j], sem_b))

    @plsc.parallel_loop(0, _HALF, step=_L)
    def fill(i):
        sl = pl.ds(i, _L)
        buf[_ONES, sl] = jnp.full((_L,), 1.0, jnp.float32)
        buf[_ZEROS, sl] = jnp.full((_L,), 0.0, jnp.float32)

    for d in lab_d:
        d.wait()

    zero_d = []
    for j in range(2):
        idx = lab2.at[j]
        zero_d.append(pltpu.async_copy(buf.at[_ZEROS], counts_sh.at[idx], sem_c))
        zero_d.append(pltpu.async_copy(buf.at[_ZEROS], sums_sh.at[idx], sem_c))
        zero_d.append(pltpu.async_copy(buf.at[_ZEROS], sqs_sh.at[idx], sem_c))

    for d in fn_d:
        d.wait()

    @plsc.parallel_loop(0, _HALF, step=_L)
    def square(i):
        sl = pl.ds(i, _L)
        for j in range(2):
            f = buf[_FN + j, sl]
            buf[_SQ2 + j, sl] = f * f

    for d in zero_d:
        d.wait()
    plsc.subcore_barrier()

    add_d = []
    for j in range(2):
        idx = lab2.at[j]
        add_d.append(pltpu.async_copy(
            buf.at[_ONES], counts_sh.at[idx], sem_a, add=True))
        add_d.append(pltpu.async_copy(
            buf.at[_FN + j], sums_sh.at[idx], sem_a, add=True))
        add_d.append(pltpu.async_copy(
            buf.at[_SQ2 + j], sqs_sh.at[idx], sem_a, add=True))
    for d in add_d:
        d.wait()
    plsc.subcore_barrier()

    gat_d = []
    for j in range(2):
        idx = lab2.at[j]
        gat_d.append(pltpu.async_copy(counts_sh.at[idx], buf.at[_CNT + j], sem_b))
        gat_d.append(pltpu.async_copy(sums_sh.at[idx], buf.at[_SUM + j], sem_b))
        gat_d.append(pltpu.async_copy(sqs_sh.at[idx], buf.at[_SQ + j], sem_b))
    for d in gat_d:
        d.wait()

    @plsc.parallel_loop(0, _HALF, step=_L)
    def norm(i):
        sl = pl.ds(i, _L)
        for j in range(2):
            cnt = buf[_CNT + j, sl]
            s = buf[_SUM + j, sl]
            q = buf[_SQ + j, sl]
            f = buf[_FN + j, sl]
            mean = s / jnp.maximum(cnt, 1.0)
            var = (q - cnt * mean * mean) / jnp.maximum(cnt - 1.0, 1.0)
            var = jnp.maximum(var, 0.0)
            y = _newton_rsqrt(jnp.maximum(var, 1e-30))
            std = var * y
            d = f - mean
            buf[_RES + j, sl] = jnp.where(cnt > 2.0, d / (std + _EPS), d / 20.0)

    for j in range(2):
        pltpu.sync_copy(buf.at[_RES + j],
                        out_hbm.at[pl.ds(sid * _CHUNK + j * _HALF, _HALF)])


@jax.jit
def _lamaface_sc(label, fn):
    mesh = plsc.VectorSubcoreMesh(core_axis_name="c", subcore_axis_name="s", num_cores=1)
    run = pl.kernel(
        _sc_body,
        out_type=jax.ShapeDtypeStruct((_BATCH,), jnp.float32),
        mesh=mesh,
        scratch_types=[
            pltpu.VMEM((2, _HALF), jnp.int32),
            pltpu.VMEM((14, _HALF), jnp.float32),
            pltpu.SemaphoreType.DMA,
            pltpu.SemaphoreType.DMA,
            pltpu.SemaphoreType.DMA,
            pltpu.VMEM_SHARED((_C_PAD,), jnp.float32),
            pltpu.VMEM_SHARED((_C_PAD,), jnp.float32),
            pltpu.VMEM_SHARED((_C_PAD,), jnp.float32),
        ],
    )
    return run(label, fn)


def kernel(feature_norm, label, kernel):
    del kernel
    res = _lamaface_sc(label, feature_norm[:, 0])
    return res[:, None]
